# baseline (speedup 1.0000x reference)
import jax
import jax.numpy as jnp
from jax import lax
from jax.experimental import pallas as pl
from jax.experimental.pallas import tpu as pltpu
from jax.experimental.pallas import tpu_sc as plsc

N = 16384
D = 256
C = 1000
CPAD = 1024
NC = 2
NS = 16
LANES = 16
NG = 8
NK = 4
KCOLS = D // NK
KV = KCOLS // LANES
ROWS_PER_GROUP = N // NG
CH = 256
NCHUNK = ROWS_PER_GROUP // CH
NBLK = 16
RB = N // NBLK
CB = CPAD // NBLK


def _norm_body(f_ref, o_ref):
    x = f_ref[...]
    ss = jnp.sum(x * x, axis=1)
    norm = jnp.sqrt(ss)
    w = (jnp.float32(1.0) / jnp.maximum(norm, jnp.float32(1e-12)))[:, None]
    xn = x * w
    o_ref[...] = xn.T.reshape(D, RB // 128, 128)


_norms = pl.pallas_call(
    _norm_body,
    grid=(NBLK,),
    in_specs=[pl.BlockSpec((RB, D), lambda i: (i, 0))],
    out_specs=pl.BlockSpec((D, RB // 128, 128), lambda i: (0, i, 0)),
    out_shape=jax.ShapeDtypeStruct((D, N // 128, 128), jnp.float32),
)


def _scatter_body(featn, labels, parts_out, cnts_out,
                  featchunk, labelbuf, acc, cnt,
                  semf0, semf1, seml0, seml1):
    c = lax.axis_index("c")
    s = lax.axis_index("s")
    g = c * 4 + s // 4
    k = s % 4
    wp = k * NG + g

    zeros16 = jnp.zeros((LANES,), jnp.float32)
    ones16 = jnp.ones((LANES,), jnp.float32)

    def _zrow(r, carry):
        for j in range(128 // LANES):
            acc[r, pl.ds(j * LANES, LANES)] = zeros16
        return carry
    lax.fori_loop(0, KCOLS * 8, _zrow, 0)

    def _zc(r, carry):
        for j in range(128 // LANES):
            cnt[r, pl.ds(j * LANES, LANES)] = zeros16
        return carry
    lax.fori_loop(0, 8, _zc, 0)

    col0 = k * KCOLS
    fsems = [semf0, semf1]
    lsems = [seml0, seml1]
    fdesc = [None, None]
    ldesc = [None, None]

    def _start(m):
        b = m % 2
        row0 = g * ROWS_PER_GROUP + m * CH
        fdesc[b] = pltpu.async_copy(
            featn.at[pl.ds(col0, KCOLS), pl.ds(row0 // 128, CH // 128), :],
            featchunk.at[b], fsems[b])
        ldesc[b] = pltpu.async_copy(
            labels.at[pl.ds(row0, CH)], labelbuf.at[b], lsems[b])

    _start(0)
    _start(1)

    for m in range(NCHUNK):
        b = m % 2
        fdesc[b].wait()
        ldesc[b].wait()

        def _acc16(t, carry):
            labs = labelbuf[b, pl.ds(t * LANES, LANES)]
            lab_hi = lax.shift_right_logical(labs, 7)
            lab_lo = lax.bitwise_and(labs, jnp.int32(127))
            tb = t // 8
            tl = (t % 8) * LANES
            for cc0 in range(0, KCOLS, 16):
                vs = [featchunk[b, cc0 + u, tb, pl.ds(tl, LANES)]
                      for u in range(16)]
                for u in range(16):
                    plsc.addupdate_scatter(
                        acc.at[pl.ds((cc0 + u) * 8, 8), :],
                        [lab_hi, lab_lo], vs[u])
            return carry
        lax.fori_loop(0, CH // LANES, _acc16, 0)

        @pl.when(k == m % NK)
        def _():
            def _c16(t, carry):
                labs = labelbuf[b, pl.ds(t * LANES, LANES)]
                lab_hi = lax.shift_right_logical(labs, 7)
                lab_lo = lax.bitwise_and(labs, jnp.int32(127))
                plsc.addupdate_scatter(cnt, [lab_hi, lab_lo], ones16)
                return carry
            lax.fori_loop(0, CH // LANES, _c16, 0)

        if m + 2 < NCHUNK:
            _start(m + 2)

    pltpu.sync_copy(acc, parts_out.at[wp])
    pltpu.sync_copy(cnt, cnts_out.at[wp])


_scatter = pl.kernel(
    _scatter_body,
    out_type=[
        jax.ShapeDtypeStruct((NK * NG, KCOLS * 8, 128), jnp.float32),
        jax.ShapeDtypeStruct((NK * NG, 8, 128), jnp.float32),
    ],
    mesh=plsc.VectorSubcoreMesh(
        core_axis_name="c", subcore_axis_name="s",
        num_cores=NC, num_subcores=NS),
    compiler_params=pltpu.CompilerParams(
        needs_layout_passes=False, use_tc_tiling_on_sc=False),
    scratch_types=[
        pltpu.VMEM((2, KCOLS, CH // 128, 128), jnp.float32),
        pltpu.VMEM((2, CH), jnp.int32),
        pltpu.VMEM((KCOLS * 8, 128), jnp.float32),
        pltpu.VMEM((8, 128), jnp.float32),
        pltpu.SemaphoreType.DMA,
        pltpu.SemaphoreType.DMA,
        pltpu.SemaphoreType.DMA,
        pltpu.SemaphoreType.DMA,
    ],
)


def _epilogue_body(parts_ref, cnts_ref, protos_ref, init_ref, out_ref):
    p = parts_ref[...]
    p = p.reshape(NK, NG, KCOLS, 8, 128)
    s_k = jnp.sum(p, axis=1)
    sums = jnp.concatenate(
        [jnp.transpose(s_k[:, :, hi, :].reshape(D, 128))
         for hi in range(8)], axis=0)
    c2 = jnp.sum(cnts_ref[...], axis=0)
    cnt = jnp.concatenate(
        [jnp.transpose(c2[hi:hi + 1, :]) for hi in range(8)],
        axis=0)
    mean = sums / jnp.maximum(cnt, jnp.float32(1.0))
    protos = protos_ref[...]
    ema = jnp.float32(0.99) * protos + jnp.float32(0.01) * mean
    present = cnt > jnp.float32(0.0)
    initd = init_ref[...] > 0
    res = jnp.where(present, jnp.where(initd, ema, mean), protos)
    out_ref[...] = res[:C]


_epilogue = pl.pallas_call(
    _epilogue_body,
    out_shape=jax.ShapeDtypeStruct((C, D), jnp.float32),
)


def kernel(features, labels, prototypes, proto_initialized):
    featn = _norms(features)
    parts, cnts = _scatter(featn, labels)
    protos_pad = jnp.pad(prototypes, ((0, CPAD - C), (0, 0)))
    init_pad = jnp.pad(proto_initialized.astype(jnp.int32),
                       (0, CPAD - C)).reshape(CPAD, 1)
    return _epilogue(parts, cnts, protos_pad, init_pad)


# speedup vs baseline: 1.0802x; 1.0802x over previous
import jax
import jax.numpy as jnp
from jax import lax
from jax.experimental import pallas as pl
from jax.experimental.pallas import tpu as pltpu
from jax.experimental.pallas import tpu_sc as plsc

N = 16384
D = 256
C = 1000
CPAD = 1024
NC = 2
NS = 16
LANES = 16
NG = 8
NK = 4
KCOLS = D // NK
KV = KCOLS // LANES
ROWS_PER_GROUP = N // NG
CH = 256
NCHUNK = ROWS_PER_GROUP // CH
NBLK = 4
RB = N // NBLK
CB = CPAD // NBLK


def _norm_body(f_ref, o_ref):
    x = f_ref[...]
    ss = jnp.sum(x * x, axis=1)
    norm = jnp.sqrt(ss)
    w = (jnp.float32(1.0) / jnp.maximum(norm, jnp.float32(1e-12)))[:, None]
    xn = x * w
    o_ref[...] = xn.T.reshape(D, RB // 128, 128)


_norms = pl.pallas_call(
    _norm_body,
    grid=(NBLK,),
    in_specs=[pl.BlockSpec((RB, D), lambda i: (i, 0))],
    out_specs=pl.BlockSpec((D, RB // 128, 128), lambda i: (0, i, 0)),
    out_shape=jax.ShapeDtypeStruct((D, N // 128, 128), jnp.float32),
)


def _scatter_body(featn, labels, parts_out, cnts_out,
                  featchunk, labelbuf, acc, cnt,
                  semf0, semf1, seml0, seml1):
    c = lax.axis_index("c")
    s = lax.axis_index("s")
    g = c * 4 + s // 4
    k = s % 4
    wp = k * NG + g

    zeros16 = jnp.zeros((LANES,), jnp.float32)
    ones16 = jnp.ones((LANES,), jnp.float32)

    def _zrow(r, carry):
        for j in range(128 // LANES):
            acc[r, pl.ds(j * LANES, LANES)] = zeros16
        return carry
    lax.fori_loop(0, KCOLS * 8, _zrow, 0)

    def _zc(r, carry):
        for j in range(128 // LANES):
            cnt[r, pl.ds(j * LANES, LANES)] = zeros16
        return carry
    lax.fori_loop(0, 8, _zc, 0)

    col0 = k * KCOLS
    fsems = [semf0, semf1]
    lsems = [seml0, seml1]
    fdesc = [None, None]
    ldesc = [None, None]

    def _start(m):
        b = m % 2
        row0 = g * ROWS_PER_GROUP + m * CH
        fdesc[b] = pltpu.async_copy(
            featn.at[pl.ds(col0, KCOLS), pl.ds(row0 // 128, CH // 128), :],
            featchunk.at[b], fsems[b])
        ldesc[b] = pltpu.async_copy(
            labels.at[pl.ds(row0, CH)], labelbuf.at[b], lsems[b])

    _start(0)
    _start(1)

    for m in range(NCHUNK):
        b = m % 2
        fdesc[b].wait()
        ldesc[b].wait()

        def _acc16(t, carry):
            labs = labelbuf[b, pl.ds(t * LANES, LANES)]
            lab_hi = lax.shift_right_logical(labs, 7)
            lab_lo = lax.bitwise_and(labs, jnp.int32(127))
            tb = t // 8
            tl = (t % 8) * LANES
            for cc0 in range(0, KCOLS, 16):
                vs = [featchunk[b, cc0 + u, tb, pl.ds(tl, LANES)]
                      for u in range(16)]
                for u in range(16):
                    plsc.addupdate_scatter(
                        acc.at[pl.ds((cc0 + u) * 8, 8), :],
                        [lab_hi, lab_lo], vs[u])
            return carry
        lax.fori_loop(0, CH // LANES, _acc16, 0)

        @pl.when(k == m % NK)
        def _():
            def _c16(t, carry):
                labs = labelbuf[b, pl.ds(t * LANES, LANES)]
                lab_hi = lax.shift_right_logical(labs, 7)
                lab_lo = lax.bitwise_and(labs, jnp.int32(127))
                plsc.addupdate_scatter(cnt, [lab_hi, lab_lo], ones16)
                return carry
            lax.fori_loop(0, CH // LANES, _c16, 0)

        if m + 2 < NCHUNK:
            _start(m + 2)

    pltpu.sync_copy(acc, parts_out.at[wp])
    pltpu.sync_copy(cnt, cnts_out.at[wp])


_scatter = pl.kernel(
    _scatter_body,
    out_type=[
        jax.ShapeDtypeStruct((NK * NG, KCOLS * 8, 128), jnp.float32),
        jax.ShapeDtypeStruct((NK * NG, 8, 128), jnp.float32),
    ],
    mesh=plsc.VectorSubcoreMesh(
        core_axis_name="c", subcore_axis_name="s",
        num_cores=NC, num_subcores=NS),
    compiler_params=pltpu.CompilerParams(
        needs_layout_passes=False, use_tc_tiling_on_sc=False),
    scratch_types=[
        pltpu.VMEM((2, KCOLS, CH // 128, 128), jnp.float32),
        pltpu.VMEM((2, CH), jnp.int32),
        pltpu.VMEM((KCOLS * 8, 128), jnp.float32),
        pltpu.VMEM((8, 128), jnp.float32),
        pltpu.SemaphoreType.DMA,
        pltpu.SemaphoreType.DMA,
        pltpu.SemaphoreType.DMA,
        pltpu.SemaphoreType.DMA,
    ],
)


def _epilogue_body(parts_ref, cnts_ref, protos_ref, init_ref, out_ref):
    p = parts_ref[...]
    p = p.reshape(NK, NG, KCOLS, 8, 128)
    s_k = jnp.sum(p, axis=1)
    sums = jnp.concatenate(
        [jnp.transpose(s_k[:, :, hi, :].reshape(D, 128))
         for hi in range(8)], axis=0)
    c2 = jnp.sum(cnts_ref[...], axis=0)
    cnt = jnp.concatenate(
        [jnp.transpose(c2[hi:hi + 1, :]) for hi in range(8)],
        axis=0)
    mean = sums / jnp.maximum(cnt, jnp.float32(1.0))
    protos = protos_ref[...]
    ema = jnp.float32(0.99) * protos + jnp.float32(0.01) * mean
    present = cnt > jnp.float32(0.0)
    initd = init_ref[...] > 0
    res = jnp.where(present, jnp.where(initd, ema, mean), protos)
    out_ref[...] = res[:C]


_epilogue = pl.pallas_call(
    _epilogue_body,
    out_shape=jax.ShapeDtypeStruct((C, D), jnp.float32),
)


def kernel(features, labels, prototypes, proto_initialized):
    featn = _norms(features)
    parts, cnts = _scatter(featn, labels)
    protos_pad = jnp.pad(prototypes, ((0, CPAD - C), (0, 0)))
    init_pad = jnp.pad(proto_initialized.astype(jnp.int32),
                       (0, CPAD - C)).reshape(CPAD, 1)
    return _epilogue(parts, cnts, protos_pad, init_pad)
